# baseline (speedup 1.0000x reference)
import functools

import jax
import jax.numpy as jnp
from jax import lax
from jax.experimental import pallas as pl
from jax.experimental.pallas import tpu as pltpu

H = 1024
NH = 16
NKV = 4
HD = 64
E = 8
TOPK = 2
I_FF = 512
T = 2048
EPS = 1e-6
THETA = 10000.0
SCALE = HD ** -0.5

BT = 256
BQ = 256
MBT = 256
NPAIR = TOPK * T
P = NPAIR + E * MBT
NB = P // MBT
HALF = HD // 2


def _qkv_body(x_ref, w_ref, wqkv_ref, out_ref):
    x = x_ref[...]
    var = jnp.mean(x * x, axis=1, keepdims=True)
    xn = x * lax.rsqrt(var + EPS) * w_ref[...]
    out_ref[...] = jnp.dot(xn, wqkv_ref[...], preferred_element_type=jnp.float32)


BK = 256


def _attn_body(q_ref, k_ref, v_ref, cq_ref, sq_ref, ck_ref, sk_ref, o_ref):
    q = q_ref[0]
    cq = cq_ref[...]
    sq = sq_ref[...]
    q1 = q[:, :HALF]
    q2 = q[:, HALF:]
    qr = jnp.concatenate([q1 * cq - q2 * sq, q2 * cq + q1 * sq], axis=1)
    qb = pl.program_id(1)
    row = qb * BQ + lax.broadcasted_iota(jnp.int32, (BQ, BK), 0)
    ci = lax.broadcasted_iota(jnp.int32, (BQ, BK), 1)

    def step(kb, carry):
        m, l, acc = carry
        koff = kb * BK
        k = k_ref[0, pl.ds(koff, BK), :]
        ck = ck_ref[pl.ds(koff, BK), :]
        sk = sk_ref[pl.ds(koff, BK), :]
        k1 = k[:, :HALF]
        k2 = k[:, HALF:]
        kr = jnp.concatenate([k1 * ck - k2 * sk, k2 * ck + k1 * sk], axis=1)
        s = lax.dot_general(qr, kr, (((1,), (1,)), ((), ())),
                            preferred_element_type=jnp.float32) * SCALE
        s = jnp.where(koff + ci <= row, s, -1e9)
        m_new = jnp.maximum(m, jnp.max(s, axis=1, keepdims=True))
        p = jnp.exp(s - m_new)
        alpha = jnp.exp(m - m_new)
        l_new = l * alpha + jnp.sum(p, axis=1, keepdims=True)
        v = v_ref[0, pl.ds(koff, BK), :]
        acc_new = acc * alpha + jnp.dot(p, v, preferred_element_type=jnp.float32)
        return (m_new, l_new, acc_new)

    m0 = jnp.full((BQ, 1), -1e30, jnp.float32)
    l0 = jnp.zeros((BQ, 1), jnp.float32)
    a0 = jnp.zeros((BQ, HD), jnp.float32)
    m, l, acc = lax.fori_loop(0, (qb + 1) * (BQ // BK), step, (m0, l0, a0))
    o_ref[0] = acc / l


def _post_body(ctx_ref, wd_ref, h_ref, ln2_ref, res_ref, x2_ref):
    attn = jnp.dot(ctx_ref[...], wd_ref[...], preferred_element_type=jnp.float32)
    res = attn + h_ref[...]
    res_ref[...] = res
    var = jnp.mean(res * res, axis=1, keepdims=True)
    x2_ref[...] = res * lax.rsqrt(var + EPS) * ln2_ref[...]


def _router_body(x2_ref, wg_ref, ti_ref, tw_ref):
    logits = jnp.dot(x2_ref[...], wg_ref[...], preferred_element_type=jnp.float32)
    col = lax.broadcasted_iota(jnp.int32, (BT, 128), 1)
    lm = jnp.where(col < E, logits, -1e30)
    m = jnp.max(lm, axis=1, keepdims=True)
    p = jnp.exp(lm - m)
    p = p / jnp.sum(p, axis=1, keepdims=True)
    v1 = jnp.max(p, axis=1, keepdims=True)
    idx1 = jnp.min(jnp.where(p == v1, col, 10000), axis=1, keepdims=True)
    p2 = jnp.where(col == idx1, -1.0, p)
    v2 = jnp.max(p2, axis=1, keepdims=True)
    idx2 = jnp.min(jnp.where(p2 == v2, col, 10000), axis=1, keepdims=True)
    denom = v1 + v2
    w1 = v1 / denom
    w2 = v2 / denom
    ti_ref[...] = jnp.where(col == 0, idx1, jnp.where(col == 1, idx2, 0)).astype(jnp.int32)
    tw_ref[...] = jnp.where(col == 0, w1, jnp.where(col == 1, w2, 0.0))


def _shared_body(x2_ref, wgu_ref, wdn_ref, out_ref):
    gu = jnp.dot(x2_ref[...].astype(jnp.bfloat16),
                 wgu_ref[...].astype(jnp.bfloat16),
                 preferred_element_type=jnp.float32)
    g = gu[:, :I_FF]
    u = gu[:, I_FF:]
    h = g * (1.0 / (1.0 + jnp.exp(-g))) * u
    out_ref[...] = jnp.dot(h.astype(jnp.bfloat16),
                           wdn_ref[...].astype(jnp.bfloat16),
                           preferred_element_type=jnp.float32)


def _moe_body(be_ref, xs_ref, wgu_ref, wdn_ref, y_ref):
    del be_ref
    gu = jnp.dot(xs_ref[...].astype(jnp.bfloat16),
                 wgu_ref[0].astype(jnp.bfloat16),
                 preferred_element_type=jnp.float32)
    g = gu[:, :I_FF]
    u = gu[:, I_FF:]
    h = g * (1.0 / (1.0 + jnp.exp(-g))) * u
    y_ref[...] = jnp.dot(h.astype(jnp.bfloat16), wdn_ref[0].astype(jnp.bfloat16),
                         preferred_element_type=jnp.float32)


def kernel(hidden_states, position_ids, ln1_w, ln2_w, W_qkv, W_dense,
           W_gate, W_moe_gu, W_moe_down, W_sh_gu, W_sh_down):
    f32 = jnp.float32
    QKV_W = NH * HD + 2 * NKV * HD

    inv = 1.0 / (THETA ** (jnp.arange(0, HALF, dtype=f32) * 2.0 / HD))
    fr = position_ids.astype(f32)[:, None] * inv[None, :]
    cos = jnp.cos(fr)
    sin = jnp.sin(fr)

    ln1 = ln1_w.reshape(1, H)
    ln2 = ln2_w.reshape(1, H)

    qkv = pl.pallas_call(
        _qkv_body,
        grid=(T // BT,),
        in_specs=[
            pl.BlockSpec((BT, H), lambda b: (b, 0)),
            pl.BlockSpec((1, H), lambda b: (0, 0)),
            pl.BlockSpec((H, QKV_W), lambda b: (0, 0)),
        ],
        out_specs=pl.BlockSpec((BT, QKV_W), lambda b: (b, 0)),
        out_shape=jax.ShapeDtypeStruct((T, QKV_W), f32),
    )(hidden_states, ln1, W_qkv)

    grp = NH // NKV
    qh = qkv[:, :NH * HD].reshape(T, NH, HD).swapaxes(0, 1)
    kh = qkv[:, NH * HD:NH * HD + NKV * HD].reshape(T, NKV, HD).swapaxes(0, 1)
    vh = qkv[:, NH * HD + NKV * HD:].reshape(T, NKV, HD).swapaxes(0, 1)
    ctx = pl.pallas_call(
        _attn_body,
        grid=(NH, T // BQ),
        in_specs=[
            pl.BlockSpec((1, BQ, HD), lambda h, qb: (h, qb, 0)),
            pl.BlockSpec((1, T, HD), lambda h, qb: (h // grp, 0, 0)),
            pl.BlockSpec((1, T, HD), lambda h, qb: (h // grp, 0, 0)),
            pl.BlockSpec((BQ, HALF), lambda h, qb: (qb, 0)),
            pl.BlockSpec((BQ, HALF), lambda h, qb: (qb, 0)),
            pl.BlockSpec((T, HALF), lambda h, qb: (0, 0)),
            pl.BlockSpec((T, HALF), lambda h, qb: (0, 0)),
        ],
        out_specs=pl.BlockSpec((1, BQ, HD), lambda h, qb: (h, qb, 0)),
        out_shape=jax.ShapeDtypeStruct((NH, T, HD), f32),
    )(qh, kh, vh, cos, sin, cos, sin)

    ctx2 = ctx.swapaxes(0, 1).reshape(T, NH * HD)
    residual, x2 = pl.pallas_call(
        _post_body,
        grid=(T // BT,),
        in_specs=[
            pl.BlockSpec((BT, NH * HD), lambda b: (b, 0)),
            pl.BlockSpec((NH * HD, H), lambda b: (0, 0)),
            pl.BlockSpec((BT, H), lambda b: (b, 0)),
            pl.BlockSpec((1, H), lambda b: (0, 0)),
        ],
        out_specs=[
            pl.BlockSpec((BT, H), lambda b: (b, 0)),
            pl.BlockSpec((BT, H), lambda b: (b, 0)),
        ],
        out_shape=[
            jax.ShapeDtypeStruct((T, H), f32),
            jax.ShapeDtypeStruct((T, H), f32),
        ],
    )(ctx2, W_dense, hidden_states, ln2)

    wg_pad = jnp.zeros((H, 128), f32).at[:, :E].set(W_gate)
    ti_pad, tw_pad = pl.pallas_call(
        _router_body,
        grid=(T // BT,),
        in_specs=[
            pl.BlockSpec((BT, H), lambda b: (b, 0)),
            pl.BlockSpec((H, 128), lambda b: (0, 0)),
        ],
        out_specs=[
            pl.BlockSpec((BT, 128), lambda b: (b, 0)),
            pl.BlockSpec((BT, 128), lambda b: (b, 0)),
        ],
        out_shape=[
            jax.ShapeDtypeStruct((T, 128), jnp.int32),
            jax.ShapeDtypeStruct((T, 128), f32),
        ],
    )(x2, wg_pad)

    shared = pl.pallas_call(
        _shared_body,
        grid=(T // BT,),
        in_specs=[
            pl.BlockSpec((BT, H), lambda b: (b, 0)),
            pl.BlockSpec((H, 2 * I_FF), lambda b: (0, 0)),
            pl.BlockSpec((I_FF, H), lambda b: (0, 0)),
        ],
        out_specs=pl.BlockSpec((BT, H), lambda b: (b, 0)),
        out_shape=jax.ShapeDtypeStruct((T, H), f32),
    )(x2, W_sh_gu, W_sh_down)

    flat_e = ti_pad[:, :TOPK].reshape(-1)
    w1 = tw_pad[:, 0]
    w2 = tw_pad[:, 1]
    order = jnp.argsort(flat_e, stable=True)
    sorted_e = flat_e[order]
    counts = jnp.sum(flat_e[None, :] == jnp.arange(E, dtype=jnp.int32)[:, None],
                     axis=1).astype(jnp.int32)
    padded = ((counts + MBT - 1) // MBT) * MBT
    pend = jnp.cumsum(padded)
    poff = pend - padded
    start = jnp.cumsum(counts) - counts
    ps = poff[sorted_e] + jnp.arange(NPAIR, dtype=jnp.int32) - start[sorted_e]
    src_tok = jnp.zeros((P,), jnp.int32).at[ps].set(
        (order // TOPK).astype(jnp.int32))
    inv_pos = jnp.zeros((NPAIR,), jnp.int32).at[order].set(ps.astype(jnp.int32))
    pos1 = inv_pos[0::TOPK]
    pos2 = inv_pos[1::TOPK]
    block_expert = jnp.minimum(
        jnp.searchsorted(pend, jnp.arange(NB, dtype=jnp.int32) * MBT,
                         side='right').astype(jnp.int32), E - 1)

    x_sorted = x2[src_tok]

    y = pl.pallas_call(
        _moe_body,
        grid_spec=pltpu.PrefetchScalarGridSpec(
            num_scalar_prefetch=1,
            grid=(NB,),
            in_specs=[
                pl.BlockSpec((MBT, H), lambda b, be: (b, 0)),
                pl.BlockSpec((1, H, 2 * I_FF), lambda b, be: (be[b], 0, 0)),
                pl.BlockSpec((1, I_FF, H), lambda b, be: (be[b], 0, 0)),
            ],
            out_specs=pl.BlockSpec((MBT, H), lambda b, be: (b, 0)),
        ),
        out_shape=jax.ShapeDtypeStruct((P, H), f32),
    )(block_expert, x_sorted, W_moe_gu, W_moe_down)

    mlp_out = shared + w1[:, None] * y[pos1] + w2[:, None] * y[pos2]

    return (mlp_out, residual)


# speedup vs baseline: 1.3572x; 1.3572x over previous
import functools

import jax
import jax.numpy as jnp
from jax import lax
from jax.experimental import pallas as pl
from jax.experimental.pallas import tpu as pltpu

H = 1024
NH = 16
NKV = 4
HD = 64
E = 8
TOPK = 2
I_FF = 512
T = 2048
EPS = 1e-6
THETA = 10000.0
SCALE = HD ** -0.5

BT = 256
BQ = 256
MBT = 256
NPAIR = TOPK * T
P = NPAIR + E * MBT
NB = P // MBT
HALF = HD // 2


def _qkv_body(x_ref, w_ref, wqkv_ref, out_ref):
    x = x_ref[...]
    var = jnp.mean(x * x, axis=1, keepdims=True)
    xn = x * lax.rsqrt(var + EPS) * w_ref[...]
    out_ref[...] = jnp.dot(xn, wqkv_ref[...], preferred_element_type=jnp.float32)


def _attn_body(q_ref, k_ref, v_ref, cq_ref, sq_ref, ck_ref, sk_ref, o_ref):
    q2h = q_ref[...]
    cq = cq_ref[...]
    sq = sq_ref[...]
    h2 = pl.program_id(0)
    parity = (h2 // 2) % 2
    k128 = k_ref[...]
    v128 = v_ref[...]
    k = jnp.where(parity == 0, k128[:, :HD], k128[:, HD:])
    v = jnp.where(parity == 0, v128[:, :HD], v128[:, HD:])
    ck = ck_ref[...]
    sk = sk_ref[...]
    k1 = k[:, :HALF]
    k2 = k[:, HALF:]
    kr = jnp.concatenate([k1 * ck - k2 * sk, k2 * ck + k1 * sk], axis=1)
    qb = pl.program_id(1)
    row = qb * BQ + lax.broadcasted_iota(jnp.int32, (BQ, T), 0)
    col = lax.broadcasted_iota(jnp.int32, (BQ, T), 1)
    neg = jnp.float32(-1e9)

    def one_head(q):
        q1 = q[:, :HALF]
        q2 = q[:, HALF:]
        qr = jnp.concatenate([q1 * cq - q2 * sq, q2 * cq + q1 * sq], axis=1)
        sc = lax.dot_general(qr, kr, (((1,), (1,)), ((), ())),
                             preferred_element_type=jnp.float32) * SCALE
        sc = jnp.where(col <= row, sc, neg)
        m = jnp.max(sc, axis=1, keepdims=True)
        p = jnp.exp(sc - m)
        p = p / jnp.sum(p, axis=1, keepdims=True)
        return jnp.dot(p, v, preferred_element_type=jnp.float32)

    oa = one_head(q2h[:, :HD])
    ob = one_head(q2h[:, HD:])
    o_ref[...] = jnp.concatenate([oa, ob], axis=1)


def _post_body(ctx_ref, wd_ref, h_ref, ln2_ref, wg_ref, wsgu_ref, wsdn_ref,
               res_ref, x2_ref, ti_ref, tw_ref, sh_ref):
    attn = jnp.dot(ctx_ref[...], wd_ref[...], preferred_element_type=jnp.float32)
    res = attn + h_ref[...]
    res_ref[...] = res
    var = jnp.mean(res * res, axis=1, keepdims=True)
    x2 = res * lax.rsqrt(var + EPS) * ln2_ref[...]
    x2_ref[...] = x2

    logits = jnp.dot(x2, wg_ref[...], preferred_element_type=jnp.float32)
    col = lax.broadcasted_iota(jnp.int32, (BT, 128), 1)
    lm = jnp.where(col < E, logits, -1e30)
    m = jnp.max(lm, axis=1, keepdims=True)
    p = jnp.exp(lm - m)
    p = p / jnp.sum(p, axis=1, keepdims=True)
    v1 = jnp.max(p, axis=1, keepdims=True)
    idx1 = jnp.min(jnp.where(p == v1, col, 10000), axis=1, keepdims=True)
    p2 = jnp.where(col == idx1, -1.0, p)
    v2 = jnp.max(p2, axis=1, keepdims=True)
    idx2 = jnp.min(jnp.where(p2 == v2, col, 10000), axis=1, keepdims=True)
    denom = v1 + v2
    w1 = v1 / denom
    w2 = v2 / denom
    ti_ref[...] = jnp.where(col == 0, idx1, jnp.where(col == 1, idx2, 0)).astype(jnp.int32)
    tw_ref[...] = jnp.where(col == 0, w1, jnp.where(col == 1, w2, 0.0))

    gu = jnp.dot(x2.astype(jnp.bfloat16), wsgu_ref[...].astype(jnp.bfloat16),
                 preferred_element_type=jnp.float32)
    g = gu[:, :I_FF]
    u = gu[:, I_FF:]
    hsh = g * (1.0 / (1.0 + jnp.exp(-g))) * u
    sh_ref[...] = jnp.dot(hsh.astype(jnp.bfloat16),
                          wsdn_ref[...].astype(jnp.bfloat16),
                          preferred_element_type=jnp.float32)


def _moe_body(be_ref, xs_ref, wgu_ref, wdn_ref, y_ref):
    del be_ref
    gu = jnp.dot(xs_ref[...].astype(jnp.bfloat16),
                 wgu_ref[0].astype(jnp.bfloat16),
                 preferred_element_type=jnp.float32)
    g = gu[:, :I_FF]
    u = gu[:, I_FF:]
    h = g * (1.0 / (1.0 + jnp.exp(-g))) * u
    y_ref[...] = jnp.dot(h.astype(jnp.bfloat16), wdn_ref[0].astype(jnp.bfloat16),
                         preferred_element_type=jnp.float32)


def kernel(hidden_states, position_ids, ln1_w, ln2_w, W_qkv, W_dense,
           W_gate, W_moe_gu, W_moe_down, W_sh_gu, W_sh_down):
    f32 = jnp.float32
    QKV_W = NH * HD + 2 * NKV * HD

    inv = 1.0 / (THETA ** (jnp.arange(0, HALF, dtype=f32) * 2.0 / HD))
    fr = position_ids.astype(f32)[:, None] * inv[None, :]
    cos = jnp.cos(fr)
    sin = jnp.sin(fr)

    ln1 = ln1_w.reshape(1, H)
    ln2 = ln2_w.reshape(1, H)

    qkv = pl.pallas_call(
        _qkv_body,
        grid=(T // BT,),
        in_specs=[
            pl.BlockSpec((BT, H), lambda b: (b, 0)),
            pl.BlockSpec((1, H), lambda b: (0, 0)),
            pl.BlockSpec((H, QKV_W), lambda b: (0, 0)),
        ],
        out_specs=pl.BlockSpec((BT, QKV_W), lambda b: (b, 0)),
        out_shape=jax.ShapeDtypeStruct((T, QKV_W), f32),
    )(hidden_states, ln1, W_qkv)

    grp = NH // NKV
    qh = qkv[:, :NH * HD].reshape(T, NH, HD).swapaxes(0, 1)
    ctx2 = pl.pallas_call(
        _attn_body,
        grid=(NH // 2, T // BQ),
        in_specs=[
            pl.BlockSpec((BQ, 2 * HD), lambda h2, qb: (qb, h2)),
            pl.BlockSpec((T, 2 * HD), lambda h2, qb: (0, NH // 2 + h2 // 4)),
            pl.BlockSpec((T, 2 * HD), lambda h2, qb: (0, (NH + NKV) // 2 + h2 // 4)),
            pl.BlockSpec((BQ, HALF), lambda h2, qb: (qb, 0)),
            pl.BlockSpec((BQ, HALF), lambda h2, qb: (qb, 0)),
            pl.BlockSpec((T, HALF), lambda h2, qb: (0, 0)),
            pl.BlockSpec((T, HALF), lambda h2, qb: (0, 0)),
        ],
        out_specs=pl.BlockSpec((BQ, 2 * HD), lambda h2, qb: (qb, h2)),
        out_shape=jax.ShapeDtypeStruct((T, NH * HD), f32),
    )(qkv, qkv, qkv, cos, sin, cos, sin)

    wg_pad = jnp.zeros((H, 128), f32).at[:, :E].set(W_gate)
    residual, x2, ti_pad, tw_pad, shared = pl.pallas_call(
        _post_body,
        grid=(T // BT,),
        in_specs=[
            pl.BlockSpec((BT, NH * HD), lambda b: (b, 0)),
            pl.BlockSpec((NH * HD, H), lambda b: (0, 0)),
            pl.BlockSpec((BT, H), lambda b: (b, 0)),
            pl.BlockSpec((1, H), lambda b: (0, 0)),
            pl.BlockSpec((H, 128), lambda b: (0, 0)),
            pl.BlockSpec((H, 2 * I_FF), lambda b: (0, 0)),
            pl.BlockSpec((I_FF, H), lambda b: (0, 0)),
        ],
        out_specs=[
            pl.BlockSpec((BT, H), lambda b: (b, 0)),
            pl.BlockSpec((BT, H), lambda b: (b, 0)),
            pl.BlockSpec((BT, 128), lambda b: (b, 0)),
            pl.BlockSpec((BT, 128), lambda b: (b, 0)),
            pl.BlockSpec((BT, H), lambda b: (b, 0)),
        ],
        out_shape=[
            jax.ShapeDtypeStruct((T, H), f32),
            jax.ShapeDtypeStruct((T, H), f32),
            jax.ShapeDtypeStruct((T, 128), jnp.int32),
            jax.ShapeDtypeStruct((T, 128), f32),
            jax.ShapeDtypeStruct((T, H), f32),
        ],
    )(ctx2, W_dense, hidden_states, ln2, wg_pad, W_sh_gu, W_sh_down)

    flat_e = ti_pad[:, :TOPK].reshape(-1)
    w1 = tw_pad[:, 0]
    w2 = tw_pad[:, 1]
    order = jnp.argsort(flat_e, stable=True)
    sorted_e = flat_e[order]
    counts = jnp.sum(flat_e[None, :] == jnp.arange(E, dtype=jnp.int32)[:, None],
                     axis=1).astype(jnp.int32)
    padded = ((counts + MBT - 1) // MBT) * MBT
    pend = jnp.cumsum(padded)
    poff = pend - padded
    start = jnp.cumsum(counts) - counts
    ps = poff[sorted_e] + jnp.arange(NPAIR, dtype=jnp.int32) - start[sorted_e]
    src_tok = jnp.zeros((P,), jnp.int32).at[ps].set(
        (order // TOPK).astype(jnp.int32))
    inv_pos = jnp.zeros((NPAIR,), jnp.int32).at[order].set(ps.astype(jnp.int32))
    pos1 = inv_pos[0::TOPK]
    pos2 = inv_pos[1::TOPK]
    block_expert = jnp.minimum(
        jnp.searchsorted(pend, jnp.arange(NB, dtype=jnp.int32) * MBT,
                         side='right').astype(jnp.int32), E - 1)

    x_sorted = x2[src_tok]

    y = pl.pallas_call(
        _moe_body,
        grid_spec=pltpu.PrefetchScalarGridSpec(
            num_scalar_prefetch=1,
            grid=(NB,),
            in_specs=[
                pl.BlockSpec((MBT, H), lambda b, be: (b, 0)),
                pl.BlockSpec((1, H, 2 * I_FF), lambda b, be: (be[b], 0, 0)),
                pl.BlockSpec((1, I_FF, H), lambda b, be: (be[b], 0, 0)),
            ],
            out_specs=pl.BlockSpec((MBT, H), lambda b, be: (b, 0)),
        ),
        out_shape=jax.ShapeDtypeStruct((P, H), f32),
    )(block_expert, x_sorted, W_moe_gu, W_moe_down)

    mlp_out = shared + w1[:, None] * y[pos1] + w2[:, None] * y[pos2]

    return (mlp_out, residual)


# speedup vs baseline: 1.4282x; 1.0523x over previous
import functools

import jax
import jax.numpy as jnp
from jax import lax
from jax.experimental import pallas as pl
from jax.experimental.pallas import tpu as pltpu

H = 1024
NH = 16
NKV = 4
HD = 64
E = 8
TOPK = 2
I_FF = 512
T = 2048
EPS = 1e-6
THETA = 10000.0
SCALE = HD ** -0.5

BT = 256
BQ = 256
MBT = 256
NPAIR = TOPK * T
P = NPAIR + E * MBT
NB = P // MBT
HALF = HD // 2


def _qkv_body(x_ref, w_ref, wqkv_ref, out_ref):
    x = x_ref[...]
    var = jnp.mean(x * x, axis=1, keepdims=True)
    xn = x * lax.rsqrt(var + EPS) * w_ref[...]
    out_ref[...] = jnp.dot(xn, wqkv_ref[...], preferred_element_type=jnp.float32)


def _attn_body(q_ref, k_ref, v_ref, cq_ref, sq_ref, ck_ref, sk_ref, o_ref):
    q2h = q_ref[...]
    cq = cq_ref[...]
    sq = sq_ref[...]
    h2 = pl.program_id(0)
    parity = (h2 // 2) % 2
    k128 = k_ref[...]
    v128 = v_ref[...]
    k = jnp.where(parity == 0, k128[:, :HD], k128[:, HD:])
    v = jnp.where(parity == 0, v128[:, :HD], v128[:, HD:])
    ck = ck_ref[...]
    sk = sk_ref[...]
    k1 = k[:, :HALF]
    k2 = k[:, HALF:]
    kr = jnp.concatenate([k1 * ck - k2 * sk, k2 * ck + k1 * sk], axis=1)
    qb = pl.program_id(1)
    row = qb * BQ + lax.broadcasted_iota(jnp.int32, (BQ, T), 0)
    col = lax.broadcasted_iota(jnp.int32, (BQ, T), 1)
    neg = jnp.float32(-1e9)

    def one_head(q):
        q1 = q[:, :HALF]
        q2 = q[:, HALF:]
        qr = jnp.concatenate([q1 * cq - q2 * sq, q2 * cq + q1 * sq], axis=1)
        sc = lax.dot_general(qr, kr, (((1,), (1,)), ((), ())),
                             preferred_element_type=jnp.float32) * SCALE
        sc = jnp.where(col <= row, sc, neg)
        m = jnp.max(sc, axis=1, keepdims=True)
        p = jnp.exp(sc - m)
        p = p / jnp.sum(p, axis=1, keepdims=True)
        return jnp.dot(p, v, preferred_element_type=jnp.float32)

    oa = one_head(q2h[:, :HD])
    ob = one_head(q2h[:, HD:])
    o_ref[...] = jnp.concatenate([oa, ob], axis=1)


def _post_body(ctx_ref, wd_ref, h_ref, ln2_ref, wg_ref, wsgu_ref, wsdn_ref,
               res_ref, x2_ref, ti_ref, tw_ref, sh_ref):
    attn = jnp.dot(ctx_ref[...], wd_ref[...], preferred_element_type=jnp.float32)
    res = attn + h_ref[...]
    res_ref[...] = res
    var = jnp.mean(res * res, axis=1, keepdims=True)
    x2 = res * lax.rsqrt(var + EPS) * ln2_ref[...]
    x2_ref[...] = x2

    logits = jnp.dot(x2, wg_ref[...], preferred_element_type=jnp.float32)
    col = lax.broadcasted_iota(jnp.int32, (BT, 128), 1)
    lm = jnp.where(col < E, logits, -1e30)
    m = jnp.max(lm, axis=1, keepdims=True)
    p = jnp.exp(lm - m)
    p = p / jnp.sum(p, axis=1, keepdims=True)
    v1 = jnp.max(p, axis=1, keepdims=True)
    idx1 = jnp.min(jnp.where(p == v1, col, 10000), axis=1, keepdims=True)
    p2 = jnp.where(col == idx1, -1.0, p)
    v2 = jnp.max(p2, axis=1, keepdims=True)
    idx2 = jnp.min(jnp.where(p2 == v2, col, 10000), axis=1, keepdims=True)
    denom = v1 + v2
    w1 = v1 / denom
    w2 = v2 / denom
    ti_ref[...] = jnp.where(col == 0, idx1, jnp.where(col == 1, idx2, 0)).astype(jnp.int32)
    tw_ref[...] = jnp.where(col == 0, w1, jnp.where(col == 1, w2, 0.0))

    gu = jnp.dot(x2.astype(jnp.bfloat16), wsgu_ref[...].astype(jnp.bfloat16),
                 preferred_element_type=jnp.float32)
    g = gu[:, :I_FF]
    u = gu[:, I_FF:]
    hsh = g * (1.0 / (1.0 + jnp.exp(-g))) * u
    sh_ref[...] = jnp.dot(hsh.astype(jnp.bfloat16),
                          wsdn_ref[...].astype(jnp.bfloat16),
                          preferred_element_type=jnp.float32)


MCH = 256


def _meta_body(e_ref, ps_ref, cnt_ref, r_scr):
    f32 = jnp.float32
    tri = (lax.broadcasted_iota(jnp.int32, (MCH, MCH), 0)
           >= lax.broadcasted_iota(jnp.int32, (MCH, MCH), 1)).astype(f32)
    lane = lax.broadcasted_iota(jnp.int32, (MCH, 128), 1)
    off = jnp.zeros((1, 128), f32)
    for c in range(NPAIR // MCH):
        e_c = e_ref[pl.ds(c * MCH, MCH), :]
        oh = (e_c == lane).astype(f32)
        cum = jnp.dot(tri, oh, preferred_element_type=f32) + off
        r_scr[pl.ds(c * MCH, MCH), :] = cum
        off = off + jnp.sum(oh, axis=0, keepdims=True)
    cnt_ref[...] = off
    padded = jnp.floor((off + (MBT - 1)) * (1.0 / MBT)).astype(jnp.int32).astype(f32) * MBT
    triL = (lax.broadcasted_iota(jnp.int32, (128, 128), 0)
            <= lax.broadcasted_iota(jnp.int32, (128, 128), 1)).astype(f32)
    pend = jnp.dot(padded, triL, preferred_element_type=f32)
    poff = pend - padded
    for c in range(NPAIR // MCH):
        e_c = e_ref[pl.ds(c * MCH, MCH), :]
        oh = (e_c == lane).astype(f32)
        cum = r_scr[pl.ds(c * MCH, MCH), :]
        vals = oh * (cum - 1.0 + poff)
        ps_ref[pl.ds(c * MCH, MCH), :] = jnp.sum(
            vals, axis=1, keepdims=True).astype(jnp.int32)


def _moe_body(be_ref, xs_ref, wgu_ref, wdn_ref, y_ref):
    del be_ref
    gu = jnp.dot(xs_ref[...].astype(jnp.bfloat16),
                 wgu_ref[0].astype(jnp.bfloat16),
                 preferred_element_type=jnp.float32)
    g = gu[:, :I_FF]
    u = gu[:, I_FF:]
    h = g * (1.0 / (1.0 + jnp.exp(-g))) * u
    y_ref[...] = jnp.dot(h.astype(jnp.bfloat16), wdn_ref[0].astype(jnp.bfloat16),
                         preferred_element_type=jnp.float32)


def kernel(hidden_states, position_ids, ln1_w, ln2_w, W_qkv, W_dense,
           W_gate, W_moe_gu, W_moe_down, W_sh_gu, W_sh_down):
    f32 = jnp.float32
    QKV_W = NH * HD + 2 * NKV * HD

    inv = 1.0 / (THETA ** (jnp.arange(0, HALF, dtype=f32) * 2.0 / HD))
    fr = position_ids.astype(f32)[:, None] * inv[None, :]
    cos = jnp.cos(fr)
    sin = jnp.sin(fr)

    ln1 = ln1_w.reshape(1, H)
    ln2 = ln2_w.reshape(1, H)

    qkv = pl.pallas_call(
        _qkv_body,
        grid=(T // BT,),
        in_specs=[
            pl.BlockSpec((BT, H), lambda b: (b, 0)),
            pl.BlockSpec((1, H), lambda b: (0, 0)),
            pl.BlockSpec((H, QKV_W), lambda b: (0, 0)),
        ],
        out_specs=pl.BlockSpec((BT, QKV_W), lambda b: (b, 0)),
        out_shape=jax.ShapeDtypeStruct((T, QKV_W), f32),
    )(hidden_states, ln1, W_qkv)

    grp = NH // NKV
    qh = qkv[:, :NH * HD].reshape(T, NH, HD).swapaxes(0, 1)
    ctx2 = pl.pallas_call(
        _attn_body,
        grid=(NH // 2, T // BQ),
        in_specs=[
            pl.BlockSpec((BQ, 2 * HD), lambda h2, qb: (qb, h2)),
            pl.BlockSpec((T, 2 * HD), lambda h2, qb: (0, NH // 2 + h2 // 4)),
            pl.BlockSpec((T, 2 * HD), lambda h2, qb: (0, (NH + NKV) // 2 + h2 // 4)),
            pl.BlockSpec((BQ, HALF), lambda h2, qb: (qb, 0)),
            pl.BlockSpec((BQ, HALF), lambda h2, qb: (qb, 0)),
            pl.BlockSpec((T, HALF), lambda h2, qb: (0, 0)),
            pl.BlockSpec((T, HALF), lambda h2, qb: (0, 0)),
        ],
        out_specs=pl.BlockSpec((BQ, 2 * HD), lambda h2, qb: (qb, h2)),
        out_shape=jax.ShapeDtypeStruct((T, NH * HD), f32),
    )(qkv, qkv, qkv, cos, sin, cos, sin)

    wg_pad = jnp.zeros((H, 128), f32).at[:, :E].set(W_gate)
    residual, x2, ti_pad, tw_pad, shared = pl.pallas_call(
        _post_body,
        grid=(T // BT,),
        in_specs=[
            pl.BlockSpec((BT, NH * HD), lambda b: (b, 0)),
            pl.BlockSpec((NH * HD, H), lambda b: (0, 0)),
            pl.BlockSpec((BT, H), lambda b: (b, 0)),
            pl.BlockSpec((1, H), lambda b: (0, 0)),
            pl.BlockSpec((H, 128), lambda b: (0, 0)),
            pl.BlockSpec((H, 2 * I_FF), lambda b: (0, 0)),
            pl.BlockSpec((I_FF, H), lambda b: (0, 0)),
        ],
        out_specs=[
            pl.BlockSpec((BT, H), lambda b: (b, 0)),
            pl.BlockSpec((BT, H), lambda b: (b, 0)),
            pl.BlockSpec((BT, 128), lambda b: (b, 0)),
            pl.BlockSpec((BT, 128), lambda b: (b, 0)),
            pl.BlockSpec((BT, H), lambda b: (b, 0)),
        ],
        out_shape=[
            jax.ShapeDtypeStruct((T, H), f32),
            jax.ShapeDtypeStruct((T, H), f32),
            jax.ShapeDtypeStruct((T, 128), jnp.int32),
            jax.ShapeDtypeStruct((T, 128), f32),
            jax.ShapeDtypeStruct((T, H), f32),
        ],
    )(ctx2, W_dense, hidden_states, ln2, wg_pad, W_sh_gu, W_sh_down)

    w1 = tw_pad[:, 0]
    w2 = tw_pad[:, 1]
    e2d = ti_pad[:, :TOPK].reshape(NPAIR, 1)
    ps2d, cnt = pl.pallas_call(
        _meta_body,
        grid=(1,),
        in_specs=[pl.BlockSpec((NPAIR, 1), lambda i: (0, 0))],
        out_specs=[
            pl.BlockSpec((NPAIR, 1), lambda i: (0, 0)),
            pl.BlockSpec((1, 128), lambda i: (0, 0)),
        ],
        out_shape=[
            jax.ShapeDtypeStruct((NPAIR, 1), jnp.int32),
            jax.ShapeDtypeStruct((1, 128), f32),
        ],
        scratch_shapes=[pltpu.VMEM((NPAIR, 128), f32)],
    )(e2d)
    ps = ps2d[:, 0]
    counts = cnt[0, :E].astype(jnp.int32)
    padded = ((counts + MBT - 1) // MBT) * MBT
    pend = jnp.cumsum(padded)
    src_tok = jnp.zeros((P,), jnp.int32).at[ps].set(
        (jnp.arange(NPAIR, dtype=jnp.int32) // TOPK))
    pos1 = ps[0::TOPK]
    pos2 = ps[1::TOPK]
    block_expert = jnp.minimum(
        jnp.searchsorted(pend, jnp.arange(NB, dtype=jnp.int32) * MBT,
                         side='right').astype(jnp.int32), E - 1)

    x_sorted = x2[src_tok]

    y = pl.pallas_call(
        _moe_body,
        grid_spec=pltpu.PrefetchScalarGridSpec(
            num_scalar_prefetch=1,
            grid=(NB,),
            in_specs=[
                pl.BlockSpec((MBT, H), lambda b, be: (b, 0)),
                pl.BlockSpec((1, H, 2 * I_FF), lambda b, be: (be[b], 0, 0)),
                pl.BlockSpec((1, I_FF, H), lambda b, be: (be[b], 0, 0)),
            ],
            out_specs=pl.BlockSpec((MBT, H), lambda b, be: (b, 0)),
        ),
        out_shape=jax.ShapeDtypeStruct((P, H), f32),
    )(block_expert, x_sorted, W_moe_gu, W_moe_down)

    mlp_out = shared + w1[:, None] * y[pos1] + w2[:, None] * y[pos2]

    return (mlp_out, residual)
